# Initial kernel scaffold; baseline (speedup 1.0000x reference)
import jax
import jax.numpy as jnp
from jax.experimental import pallas as pl


def kernel(prompt_embedding, W_to3d, b_to3d, W_s1, b_s1, W_s2, b_s2, centers1, portal1, W_r1, b_r1, centers2, portal2, W_r2, b_r2, centers3, W_r3, b_r3, temperature):
    raise NotImplementedError("write your pallas kernel here")


# speedup vs baseline: 1.1138x; 1.1138x over previous
import functools

import jax
import jax.numpy as jnp
from jax.experimental import pallas as pl
from jax.experimental.pallas import tpu as pltpu

B = 8192
E = 4096
S = 64
L1, L2, L3 = 4, 16, 64
BM = 256
F32 = jnp.float32
BF16 = jnp.bfloat16


def _dsq(pos_cols, ct_ref):
    d = None
    for i in range(3):
        e = pos_cols[i] - ct_ref[i:i + 1, :].astype(F32)
        e = e * e
        d = e if d is None else d + e
    return d


def _argmax_lane(vals, total):
    m = jnp.max(vals, axis=1, keepdims=True)
    io = jax.lax.broadcasted_iota(jnp.int32, vals.shape, 1)
    sel = jnp.where(vals == m, io, total)
    return jnp.min(sel, axis=1, keepdims=True), io


def _portal_cols(p_ref, pos_cols_b, pn, total):
    out_cols = []
    for i in range(3):
        q = None
        for j in range(4):
            coef = p_ref[j:j + 1, i * total:(i + 1) * total].astype(F32)
            term = coef * pos_cols_b[j] if j < 3 else coef
            q = term if q is None else q + term
        out_cols.append(jnp.sum(pn * q, axis=1, keepdims=True))
    return out_cols


def _rep4(lp, total):
    return jnp.concatenate(
        [jnp.broadcast_to(lp[:, k:k + 1], (lp.shape[0], 4))
         for k in range(total)], axis=1)


def _router_kernel(den_ref, xb_ref, pos_ref, ws1_ref, bs1_ref, ws2_ref,
                   bs2_ref, wr_ref, br_ref,
                   c1t_ref, c2t_ref, c3t_ref, p1p_ref, p2p_ref,
                   eid_ref, p3_ref, route_ref, conf_ref):
    xb = xb_ref[...]
    pos = pos_ref[...]

    h = jnp.dot(xb, ws1_ref[...], preferred_element_type=F32) + bs1_ref[...]
    h = 0.5 * h * (1.0 + jax.lax.erf(h * 0.7071067811865476))
    spectral = jnp.tanh(
        jnp.dot(h.astype(BF16), ws2_ref[...], preferred_element_type=F32)
        + bs2_ref[...])

    nall = jax.nn.sigmoid(
        jnp.dot(spectral.astype(BF16), wr_ref[...],
                preferred_element_type=F32) + br_ref[...])
    n1 = nall[:, 0:L1]
    n2 = nall[:, 32:32 + L2]
    n3 = nall[:, 64:64 + L3]
    den = den_ref[0, 0]

    pos_cols = [pos[:, i:i + 1] for i in range(3)]

    lg1 = -_dsq(pos_cols, c1t_ref) / den
    c1, io1 = _argmax_lane(lg1, L1)
    p1 = jnp.where(io1 == c1, 1.0, 0.0) * n1
    p1n = p1 / (jnp.sum(p1, axis=1, keepdims=True) + 1e-8)
    lp1 = jnp.log(p1n + 1e-10)
    pos_b = [c.astype(BF16).astype(F32) for c in pos_cols]
    pos1_cols = _portal_cols(p1p_ref, pos_b, p1n, L1)

    lg2 = -_dsq(pos1_cols, c2t_ref) / den + _rep4(lp1, L1)
    c2, io2 = _argmax_lane(lg2, L2)
    p2 = jnp.where(io2 == c2, 1.0, 0.0) * n2
    p2n = p2 / (jnp.sum(p2, axis=1, keepdims=True) + 1e-8)
    lp2 = jnp.log(p2n + 1e-10)
    pos1_b = [c.astype(BF16).astype(F32) for c in pos1_cols]
    pos2_cols = _portal_cols(p2p_ref, pos1_b, p2n, L2)

    lg3 = -_dsq(pos2_cols, c3t_ref) / den + _rep4(lp2, L2)
    c3, io3 = _argmax_lane(lg3, L3)
    p3 = jnp.where(io3 == c3, 1.0, 0.0) * n3
    p3n = p3 / (jnp.sum(p3, axis=1, keepdims=True) + 1e-8)

    conf = jnp.max(p3n, axis=1, keepdims=True)
    eid = jnp.min(jnp.where(p3n == conf, io3, L3), axis=1, keepdims=True)

    eid_ref[...] = eid
    p3_ref[...] = p3n
    route_ref[...] = jnp.concatenate([c1, c2, c3], axis=1)
    conf_ref[...] = conf


def kernel(prompt_embedding, W_to3d, b_to3d, W_s1, b_s1, W_s2, b_s2,
           centers1, portal1, W_r1, b_r1,
           centers2, portal2, W_r2, b_r2,
           centers3, W_r3, b_r3, temperature):
    pos_3d = prompt_embedding @ W_to3d + b_to3d
    xb = prompt_embedding.astype(BF16)
    den = (2.0 * temperature ** 2 + 1e-8).astype(F32).reshape(1, 1)
    ws1 = W_s1.astype(BF16)
    ws2 = W_s2.astype(BF16)
    bs1 = b_s1.reshape(1, E)
    bs2 = b_s2.reshape(1, S)
    wr = jnp.concatenate(
        [W_r1, jnp.zeros((S, 32 - L1), F32),
         W_r2, jnp.zeros((S, 32 - L2), F32), W_r3], axis=1).astype(BF16)
    br = jnp.concatenate(
        [b_r1, jnp.zeros((32 - L1,), F32),
         b_r2, jnp.zeros((32 - L2,), F32), b_r3]).reshape(1, 128)
    c1t = centers1.T
    c2t = centers2.T
    c3t = centers3.T
    p1p = portal1.transpose(2, 1, 0).reshape(4, 3 * L1).astype(BF16)
    p2p = portal2.transpose(2, 1, 0).reshape(4, 3 * L2).astype(BF16)

    def fixed(shape):
        return pl.BlockSpec(shape, lambda i: (0, 0))

    grid = (B // BM,)
    eid, p3, route, conf = pl.pallas_call(
        _router_kernel,
        grid=grid,
        in_specs=[
            fixed((1, 1)),
            pl.BlockSpec((BM, E), lambda i: (i, 0)),
            pl.BlockSpec((BM, 3), lambda i: (i, 0)),
            fixed((E, E)),
            fixed((1, E)),
            fixed((E, S)),
            fixed((1, S)),
            fixed((S, 128)),
            fixed((1, 128)),
            fixed((3, L1)),
            fixed((3, L2)),
            fixed((3, L3)),
            fixed((4, 3 * L1)),
            fixed((4, 3 * L2)),
        ],
        out_specs=[
            pl.BlockSpec((BM, 1), lambda i: (i, 0)),
            pl.BlockSpec((BM, L3), lambda i: (i, 0)),
            pl.BlockSpec((BM, 3), lambda i: (i, 0)),
            pl.BlockSpec((BM, 1), lambda i: (i, 0)),
        ],
        out_shape=[
            jax.ShapeDtypeStruct((B, 1), jnp.int32),
            jax.ShapeDtypeStruct((B, L3), F32),
            jax.ShapeDtypeStruct((B, 3), jnp.int32),
            jax.ShapeDtypeStruct((B, 1), F32),
        ],
        compiler_params=pltpu.CompilerParams(
            dimension_semantics=("arbitrary",)),
    )(den, xb, pos_3d, ws1, bs1, ws2, bs2, wr, br,
      c1t, c2t, c3t, p1p, p2p)

    return (eid.reshape(B), p3, route, conf.reshape(B))


# speedup vs baseline: 1.7252x; 1.5489x over previous
import functools

import jax
import jax.numpy as jnp
from jax.experimental import pallas as pl
from jax.experimental.pallas import tpu as pltpu

B = 8192
E = 4096
S = 64
L1, L2, L3 = 4, 16, 64
BM = 256
F32 = jnp.float32
BF16 = jnp.bfloat16


def _dsq(pos_cols, ct_ref):
    d = None
    for i in range(3):
        e = pos_cols[i] - ct_ref[i:i + 1, :].astype(F32)
        e = e * e
        d = e if d is None else d + e
    return d


def _argmax_lane(vals, total):
    m = jnp.max(vals, axis=1, keepdims=True)
    io = jax.lax.broadcasted_iota(jnp.int32, vals.shape, 1)
    sel = jnp.where(vals == m, io, total)
    return jnp.min(sel, axis=1, keepdims=True), io


def _portal_cols(p_ref, pos_cols_b, pn, total):
    out_cols = []
    for i in range(3):
        q = None
        for j in range(4):
            coef = p_ref[j:j + 1, i * total:(i + 1) * total].astype(F32)
            term = coef * pos_cols_b[j] if j < 3 else coef
            q = term if q is None else q + term
        out_cols.append(jnp.sum(pn * q, axis=1, keepdims=True))
    return out_cols


def _rep4(lp, total):
    return jnp.concatenate(
        [jnp.broadcast_to(lp[:, k:k + 1], (lp.shape[0], 4))
         for k in range(total)], axis=1)


def _router_kernel(den_ref, x_ref, pos_ref, ws1_ref, bs1_ref, ws2_ref,
                   bs2_ref, wr_ref, br_ref,
                   c1t_ref, c2t_ref, c3t_ref, p1p_ref, p2p_ref,
                   eid_ref, p3_ref, route_ref, conf_ref):
    x8 = x_ref[...].astype(jnp.float8_e4m3fn)
    pos = pos_ref[...]

    h = (jnp.dot(x8, ws1_ref[...], preferred_element_type=F32) * 0.015625
         + bs1_ref[...])
    h = 0.5 * h * (1.0 + jax.lax.erf(h * 0.7071067811865476))
    spectral = jnp.tanh(
        jnp.dot(h.astype(BF16), ws2_ref[...], preferred_element_type=F32)
        + bs2_ref[...])

    nall = jax.nn.sigmoid(
        jnp.dot(spectral.astype(BF16), wr_ref[...],
                preferred_element_type=F32) + br_ref[...])
    n1 = nall[:, 0:L1]
    n2 = nall[:, 32:32 + L2]
    n3 = nall[:, 64:64 + L3]
    den = den_ref[0, 0]

    pos_cols = [pos[:, i:i + 1] for i in range(3)]

    lg1 = -_dsq(pos_cols, c1t_ref) / den
    c1, io1 = _argmax_lane(lg1, L1)
    p1 = jnp.where(io1 == c1, 1.0, 0.0) * n1
    p1n = p1 / (jnp.sum(p1, axis=1, keepdims=True) + 1e-8)
    lp1 = jnp.log(p1n + 1e-10)
    pos_b = [c.astype(BF16).astype(F32) for c in pos_cols]
    pos1_cols = _portal_cols(p1p_ref, pos_b, p1n, L1)

    lg2 = -_dsq(pos1_cols, c2t_ref) / den + _rep4(lp1, L1)
    c2, io2 = _argmax_lane(lg2, L2)
    p2 = jnp.where(io2 == c2, 1.0, 0.0) * n2
    p2n = p2 / (jnp.sum(p2, axis=1, keepdims=True) + 1e-8)
    lp2 = jnp.log(p2n + 1e-10)
    pos1_b = [c.astype(BF16).astype(F32) for c in pos1_cols]
    pos2_cols = _portal_cols(p2p_ref, pos1_b, p2n, L2)

    lg3 = -_dsq(pos2_cols, c3t_ref) / den + _rep4(lp2, L2)
    c3, io3 = _argmax_lane(lg3, L3)
    p3 = jnp.where(io3 == c3, 1.0, 0.0) * n3
    p3n = p3 / (jnp.sum(p3, axis=1, keepdims=True) + 1e-8)

    conf = jnp.max(p3n, axis=1, keepdims=True)
    eid = jnp.min(jnp.where(p3n == conf, io3, L3), axis=1, keepdims=True)

    eid_ref[...] = eid
    p3_ref[...] = p3n
    route_ref[...] = jnp.concatenate([c1, c2, c3], axis=1)
    conf_ref[...] = conf


def kernel(prompt_embedding, W_to3d, b_to3d, W_s1, b_s1, W_s2, b_s2,
           centers1, portal1, W_r1, b_r1,
           centers2, portal2, W_r2, b_r2,
           centers3, W_r3, b_r3, temperature):
    pos_3d = prompt_embedding @ W_to3d + b_to3d
    den = (2.0 * temperature ** 2 + 1e-8).astype(F32).reshape(1, 1)
    ws1 = (W_s1 * 64.0).astype(jnp.float8_e4m3fn)
    ws2 = W_s2.astype(BF16)
    bs1 = b_s1.reshape(1, E)
    bs2 = b_s2.reshape(1, S)
    wr = jnp.concatenate(
        [W_r1, jnp.zeros((S, 32 - L1), F32),
         W_r2, jnp.zeros((S, 32 - L2), F32), W_r3], axis=1).astype(BF16)
    br = jnp.concatenate(
        [b_r1, jnp.zeros((32 - L1,), F32),
         b_r2, jnp.zeros((32 - L2,), F32), b_r3]).reshape(1, 128)
    c1t = centers1.T
    c2t = centers2.T
    c3t = centers3.T
    p1p = portal1.transpose(2, 1, 0).reshape(4, 3 * L1).astype(BF16)
    p2p = portal2.transpose(2, 1, 0).reshape(4, 3 * L2).astype(BF16)

    def fixed(shape):
        return pl.BlockSpec(shape, lambda i: (0, 0))

    grid = (B // BM,)
    eid, p3, route, conf = pl.pallas_call(
        _router_kernel,
        grid=grid,
        in_specs=[
            fixed((1, 1)),
            pl.BlockSpec((BM, E), lambda i: (i, 0)),
            pl.BlockSpec((BM, 3), lambda i: (i, 0)),
            fixed((E, E)),
            fixed((1, E)),
            fixed((E, S)),
            fixed((1, S)),
            fixed((S, 128)),
            fixed((1, 128)),
            fixed((3, L1)),
            fixed((3, L2)),
            fixed((3, L3)),
            fixed((4, 3 * L1)),
            fixed((4, 3 * L2)),
        ],
        out_specs=[
            pl.BlockSpec((BM, 1), lambda i: (i, 0)),
            pl.BlockSpec((BM, L3), lambda i: (i, 0)),
            pl.BlockSpec((BM, 3), lambda i: (i, 0)),
            pl.BlockSpec((BM, 1), lambda i: (i, 0)),
        ],
        out_shape=[
            jax.ShapeDtypeStruct((B, 1), jnp.int32),
            jax.ShapeDtypeStruct((B, L3), F32),
            jax.ShapeDtypeStruct((B, 3), jnp.int32),
            jax.ShapeDtypeStruct((B, 1), F32),
        ],
        compiler_params=pltpu.CompilerParams(
            dimension_semantics=("arbitrary",)),
    )(den, prompt_embedding, pos_3d, ws1, bs1, ws2, bs2, wr, br,
      c1t, c2t, c3t, p1p, p2p)

    return (eid.reshape(B), p3, route, conf.reshape(B))


# speedup vs baseline: 1.7824x; 1.0332x over previous
import functools

import jax
import jax.numpy as jnp
from jax.experimental import pallas as pl
from jax.experimental.pallas import tpu as pltpu

B = 8192
E = 4096
S = 64
L1, L2, L3 = 4, 16, 64
BM = 512
F32 = jnp.float32
BF16 = jnp.bfloat16


def _dsq(pos_cols, ct_ref):
    d = None
    for i in range(3):
        e = pos_cols[i] - ct_ref[i:i + 1, :].astype(F32)
        e = e * e
        d = e if d is None else d + e
    return d


def _argmax_lane(vals, total):
    m = jnp.max(vals, axis=1, keepdims=True)
    io = jax.lax.broadcasted_iota(jnp.int32, vals.shape, 1)
    sel = jnp.where(vals == m, io, total)
    return jnp.min(sel, axis=1, keepdims=True), io


def _portal_cols(p_ref, pos_cols_b, pn, total):
    out_cols = []
    for i in range(3):
        q = None
        for j in range(4):
            coef = p_ref[j:j + 1, i * total:(i + 1) * total].astype(F32)
            term = coef * pos_cols_b[j] if j < 3 else coef
            q = term if q is None else q + term
        out_cols.append(jnp.sum(pn * q, axis=1, keepdims=True))
    return out_cols


def _rep4(lp, total):
    return jnp.concatenate(
        [jnp.broadcast_to(lp[:, k:k + 1], (lp.shape[0], 4))
         for k in range(total)], axis=1)


def _router_kernel(den_ref, x_ref, pos_ref, ws1_ref, bs1_ref, ws2_ref,
                   bs2_ref, wr_ref, br_ref,
                   c1t_ref, c2t_ref, c3t_ref, p1p_ref, p2p_ref,
                   eid_ref, p3_ref, route_ref, conf_ref):
    x8 = x_ref[...].astype(jnp.float8_e4m3fn)
    pos = pos_ref[...]

    h = (jnp.dot(x8, ws1_ref[...], preferred_element_type=F32) * 0.015625
         + bs1_ref[...])
    h = 0.5 * h * (1.0 + jax.lax.erf(h * 0.7071067811865476))
    spectral = jnp.tanh(
        jnp.dot(h.astype(BF16), ws2_ref[...], preferred_element_type=F32)
        + bs2_ref[...])

    nall = jax.nn.sigmoid(
        jnp.dot(spectral.astype(BF16), wr_ref[...],
                preferred_element_type=F32) + br_ref[...])
    n1 = nall[:, 0:L1]
    n2 = nall[:, 32:32 + L2]
    n3 = nall[:, 64:64 + L3]
    den = den_ref[0, 0]

    pos_cols = [pos[:, i:i + 1] for i in range(3)]

    lg1 = -_dsq(pos_cols, c1t_ref) / den
    c1, io1 = _argmax_lane(lg1, L1)
    p1 = jnp.where(io1 == c1, 1.0, 0.0) * n1
    p1n = p1 / (jnp.sum(p1, axis=1, keepdims=True) + 1e-8)
    lp1 = jnp.log(p1n + 1e-10)
    pos_b = [c.astype(BF16).astype(F32) for c in pos_cols]
    pos1_cols = _portal_cols(p1p_ref, pos_b, p1n, L1)

    lg2 = -_dsq(pos1_cols, c2t_ref) / den + _rep4(lp1, L1)
    c2, io2 = _argmax_lane(lg2, L2)
    p2 = jnp.where(io2 == c2, 1.0, 0.0) * n2
    p2n = p2 / (jnp.sum(p2, axis=1, keepdims=True) + 1e-8)
    lp2 = jnp.log(p2n + 1e-10)
    pos1_b = [c.astype(BF16).astype(F32) for c in pos1_cols]
    pos2_cols = _portal_cols(p2p_ref, pos1_b, p2n, L2)

    lg3 = -_dsq(pos2_cols, c3t_ref) / den + _rep4(lp2, L2)
    c3, io3 = _argmax_lane(lg3, L3)
    p3 = jnp.where(io3 == c3, 1.0, 0.0) * n3
    p3n = p3 / (jnp.sum(p3, axis=1, keepdims=True) + 1e-8)

    conf = jnp.max(p3n, axis=1, keepdims=True)
    eid = jnp.min(jnp.where(p3n == conf, io3, L3), axis=1, keepdims=True)

    eid_ref[...] = eid
    p3_ref[...] = p3n
    route_ref[...] = jnp.concatenate([c1, c2, c3], axis=1)
    conf_ref[...] = conf


def kernel(prompt_embedding, W_to3d, b_to3d, W_s1, b_s1, W_s2, b_s2,
           centers1, portal1, W_r1, b_r1,
           centers2, portal2, W_r2, b_r2,
           centers3, W_r3, b_r3, temperature):
    pos_3d = prompt_embedding @ W_to3d + b_to3d
    den = (2.0 * temperature ** 2 + 1e-8).astype(F32).reshape(1, 1)
    ws1 = (W_s1 * 64.0).astype(jnp.float8_e4m3fn)
    ws2 = W_s2.astype(BF16)
    bs1 = b_s1.reshape(1, E)
    bs2 = b_s2.reshape(1, S)
    wr = jnp.concatenate(
        [W_r1, jnp.zeros((S, 32 - L1), F32),
         W_r2, jnp.zeros((S, 32 - L2), F32), W_r3], axis=1).astype(BF16)
    br = jnp.concatenate(
        [b_r1, jnp.zeros((32 - L1,), F32),
         b_r2, jnp.zeros((32 - L2,), F32), b_r3]).reshape(1, 128)
    c1t = centers1.T
    c2t = centers2.T
    c3t = centers3.T
    p1p = portal1.transpose(2, 1, 0).reshape(4, 3 * L1).astype(BF16)
    p2p = portal2.transpose(2, 1, 0).reshape(4, 3 * L2).astype(BF16)

    def fixed(shape):
        return pl.BlockSpec(shape, lambda i: (0, 0))

    grid = (B // BM,)
    eid, p3, route, conf = pl.pallas_call(
        _router_kernel,
        grid=grid,
        in_specs=[
            fixed((1, 1)),
            pl.BlockSpec((BM, E), lambda i: (i, 0)),
            pl.BlockSpec((BM, 3), lambda i: (i, 0)),
            fixed((E, E)),
            fixed((1, E)),
            fixed((E, S)),
            fixed((1, S)),
            fixed((S, 128)),
            fixed((1, 128)),
            fixed((3, L1)),
            fixed((3, L2)),
            fixed((3, L3)),
            fixed((4, 3 * L1)),
            fixed((4, 3 * L2)),
        ],
        out_specs=[
            pl.BlockSpec((BM, 1), lambda i: (i, 0)),
            pl.BlockSpec((BM, L3), lambda i: (i, 0)),
            pl.BlockSpec((BM, 3), lambda i: (i, 0)),
            pl.BlockSpec((BM, 1), lambda i: (i, 0)),
        ],
        out_shape=[
            jax.ShapeDtypeStruct((B, 1), jnp.int32),
            jax.ShapeDtypeStruct((B, L3), F32),
            jax.ShapeDtypeStruct((B, 3), jnp.int32),
            jax.ShapeDtypeStruct((B, 1), F32),
        ],
        compiler_params=pltpu.CompilerParams(
            dimension_semantics=("arbitrary",)),
    )(den, prompt_embedding, pos_3d, ws1, bs1, ws2, bs2, wr, br,
      c1t, c2t, c3t, p1p, p2p)

    return (eid.reshape(B), p3, route, conf.reshape(B))


# speedup vs baseline: 2.0001x; 1.1221x over previous
import jax
import jax.numpy as jnp
from jax.experimental import pallas as pl
from jax.experimental.pallas import tpu as pltpu

B = 8192
E = 4096
S = 64
L1, L2, L3 = 4, 16, 64
BM = 512
F32 = jnp.float32
BF16 = jnp.bfloat16


def _spectral_kernel(x_ref, ws1_ref, bs1_ref, ws2_ref, bs2_ref,
                     wr_ref, br_ref, nall_ref):
    x8 = x_ref[...].astype(jnp.float8_e4m3fn)
    h = (jnp.dot(x8, ws1_ref[...], preferred_element_type=F32) * 0.015625
         + bs1_ref[...])
    h = 0.5 * h * (1.0 + jax.lax.erf(h * 0.7071067811865476))
    spectral = jnp.tanh(
        jnp.dot(h.astype(BF16), ws2_ref[...], preferred_element_type=F32)
        + bs2_ref[...])
    nall_ref[...] = jax.nn.sigmoid(
        jnp.dot(spectral.astype(BF16), wr_ref[...],
                preferred_element_type=F32) + br_ref[...])


def _dsq(pos_cols, ct_ref):
    d = None
    for i in range(3):
        e = pos_cols[i] - ct_ref[i:i + 1, :].astype(F32)
        e = e * e
        d = e if d is None else d + e
    return d


def _argmax_lane(vals, total):
    m = jnp.max(vals, axis=1, keepdims=True)
    io = jax.lax.broadcasted_iota(jnp.int32, vals.shape, 1)
    sel = jnp.where(vals == m, io, total)
    return jnp.min(sel, axis=1, keepdims=True), io


def _portal_cols(p_ref, pos_cols_b, pn, total):
    out_cols = []
    for i in range(3):
        q = None
        for j in range(4):
            coef = p_ref[j:j + 1, i * total:(i + 1) * total].astype(F32)
            term = coef * pos_cols_b[j] if j < 3 else coef
            q = term if q is None else q + term
        out_cols.append(jnp.sum(pn * q, axis=1, keepdims=True))
    return out_cols


def _rep4(lp, total):
    return jnp.concatenate(
        [jnp.broadcast_to(lp[:, k:k + 1], (lp.shape[0], 4))
         for k in range(total)], axis=1)


def _routing_kernel(den_ref, pos_ref, nall_ref,
                    c1t_ref, c2t_ref, c3t_ref, p1p_ref, p2p_ref,
                    eid_ref, p3_ref, route_ref, conf_ref):
    pos = pos_ref[...]
    nall = nall_ref[...]
    n1 = nall[:, 0:L1]
    n2 = nall[:, 32:32 + L2]
    n3 = nall[:, 64:64 + L3]
    den = den_ref[0, 0]

    pos_cols = [pos[:, i:i + 1] for i in range(3)]

    lg1 = -_dsq(pos_cols, c1t_ref) / den
    c1, io1 = _argmax_lane(lg1, L1)
    p1 = jnp.where(io1 == c1, 1.0, 0.0) * n1
    p1n = p1 / (jnp.sum(p1, axis=1, keepdims=True) + 1e-8)
    lp1 = jnp.log(p1n + 1e-10)
    pos_b = [c.astype(BF16).astype(F32) for c in pos_cols]
    pos1_cols = _portal_cols(p1p_ref, pos_b, p1n, L1)

    lg2 = -_dsq(pos1_cols, c2t_ref) / den + _rep4(lp1, L1)
    c2, io2 = _argmax_lane(lg2, L2)
    p2 = jnp.where(io2 == c2, 1.0, 0.0) * n2
    p2n = p2 / (jnp.sum(p2, axis=1, keepdims=True) + 1e-8)
    lp2 = jnp.log(p2n + 1e-10)
    pos1_b = [c.astype(BF16).astype(F32) for c in pos1_cols]
    pos2_cols = _portal_cols(p2p_ref, pos1_b, p2n, L2)

    lg3 = -_dsq(pos2_cols, c3t_ref) / den + _rep4(lp2, L2)
    c3, io3 = _argmax_lane(lg3, L3)
    p3 = jnp.where(io3 == c3, 1.0, 0.0) * n3
    p3n = p3 / (jnp.sum(p3, axis=1, keepdims=True) + 1e-8)

    conf = jnp.max(p3n, axis=1, keepdims=True)
    eid = jnp.min(jnp.where(p3n == conf, io3, L3), axis=1, keepdims=True)

    eid_ref[...] = eid
    p3_ref[...] = p3n
    route_ref[...] = jnp.concatenate([c1, c2, c3], axis=1)
    conf_ref[...] = conf


def kernel(prompt_embedding, W_to3d, b_to3d, W_s1, b_s1, W_s2, b_s2,
           centers1, portal1, W_r1, b_r1,
           centers2, portal2, W_r2, b_r2,
           centers3, W_r3, b_r3, temperature):
    pos_3d = prompt_embedding @ W_to3d + b_to3d
    den = (2.0 * temperature ** 2 + 1e-8).astype(F32).reshape(1, 1)
    ws1 = (W_s1 * 64.0).astype(jnp.float8_e4m3fn)
    ws2 = W_s2.astype(BF16)
    bs1 = b_s1.reshape(1, E)
    bs2 = b_s2.reshape(1, S)
    wr = jnp.concatenate(
        [W_r1, jnp.zeros((S, 32 - L1), F32),
         W_r2, jnp.zeros((S, 32 - L2), F32), W_r3], axis=1).astype(BF16)
    br = jnp.concatenate(
        [b_r1, jnp.zeros((32 - L1,), F32),
         b_r2, jnp.zeros((32 - L2,), F32), b_r3]).reshape(1, 128)
    c1t = centers1.T
    c2t = centers2.T
    c3t = centers3.T
    p1p = portal1.transpose(2, 1, 0).reshape(4, 3 * L1).astype(BF16)
    p2p = portal2.transpose(2, 1, 0).reshape(4, 3 * L2).astype(BF16)

    def fixed(shape):
        return pl.BlockSpec(shape, lambda i: (0, 0))

    nall = pl.pallas_call(
        _spectral_kernel,
        grid=(B // BM,),
        in_specs=[
            pl.BlockSpec((BM, E), lambda i: (i, 0)),
            fixed((E, E)),
            fixed((1, E)),
            fixed((E, S)),
            fixed((1, S)),
            fixed((S, 128)),
            fixed((1, 128)),
        ],
        out_specs=pl.BlockSpec((BM, 128), lambda i: (i, 0)),
        out_shape=jax.ShapeDtypeStruct((B, 128), F32),
        compiler_params=pltpu.CompilerParams(
            dimension_semantics=("arbitrary",)),
    )(prompt_embedding, ws1, bs1, ws2, bs2, wr, br)

    RB = 2048
    eid, p3, route, conf = pl.pallas_call(
        _routing_kernel,
        grid=(B // RB,),
        in_specs=[
            fixed((1, 1)),
            pl.BlockSpec((RB, 3), lambda i: (i, 0)),
            pl.BlockSpec((RB, 128), lambda i: (i, 0)),
            fixed((3, L1)),
            fixed((3, L2)),
            fixed((3, L3)),
            fixed((4, 3 * L1)),
            fixed((4, 3 * L2)),
        ],
        out_specs=[
            pl.BlockSpec((RB, 1), lambda i: (i, 0)),
            pl.BlockSpec((RB, L3), lambda i: (i, 0)),
            pl.BlockSpec((RB, 3), lambda i: (i, 0)),
            pl.BlockSpec((RB, 1), lambda i: (i, 0)),
        ],
        out_shape=[
            jax.ShapeDtypeStruct((B, 1), jnp.int32),
            jax.ShapeDtypeStruct((B, L3), F32),
            jax.ShapeDtypeStruct((B, 3), jnp.int32),
            jax.ShapeDtypeStruct((B, 1), F32),
        ],
        compiler_params=pltpu.CompilerParams(
            dimension_semantics=("arbitrary",)),
    )(den, pos_3d, nall, c1t, c2t, c3t, p1p, p2p)

    return (eid.reshape(B), p3, route, conf.reshape(B))
